# baseline (speedup 1.0000x reference)
import functools

import jax
import jax.numpy as jnp
from jax import lax
from jax.experimental import pallas as pl
from jax.experimental.pallas import tpu as pltpu
from jax.experimental.pallas import tpu_sc as plsc

N = 16384
K = 8192
D = 256
TN = 512
GRID = N // TN

_CHUNK = 2736
_B0 = 21 * 128
_B1 = 42 * 128
_INF = float("inf")


def _round_f32_to_bf16_rte(v):
    u = lax.bitcast_convert_type(v, jnp.uint32)
    r = (u + jnp.uint32(0x7FFF) + ((u >> jnp.uint32(16)) & jnp.uint32(1))) \
        & jnp.uint32(0xFFFF0000)
    return lax.bitcast_convert_type(r, jnp.float32)


def _lane_iota(tn):
    return lax.broadcasted_iota(jnp.int32, (tn, 128), 1)


def _dist_argmax_kernel(x_ref, et_ref, a2_ref, c2_ref, krf_ref,
                        ind_ref, s_ref):
    x = x_ref[...]
    et = et_ref[...]
    b = jnp.dot(x, et, preferred_element_type=jnp.float32)
    s_ref[...] = (a2_ref[...] - b) + c2_ref[...]

    li = _lane_iota(TN)
    col0 = s_ref[:, _B0:_B0 + 128]
    col1 = s_ref[:, _B1:_B1 + 128]
    m0 = jnp.minimum(
        jnp.min(s_ref[:, 0:_B0], axis=1),
        jnp.min(jnp.where(li < 48, col0, _INF), axis=1))
    m1 = jnp.minimum(
        jnp.min(s_ref[:, _B0 + 128:_B1], axis=1),
        jnp.minimum(
            jnp.min(jnp.where(li >= 48, col0, _INF), axis=1),
            jnp.min(jnp.where(li < 96, col1, _INF), axis=1)))
    m2 = jnp.minimum(
        jnp.min(s_ref[:, _B1 + 128:K], axis=1),
        jnp.min(jnp.where(li >= 96, col1, _INF), axis=1))

    acc = _round_f32_to_bf16_rte(m0)
    chsel = jnp.zeros(m0.shape, jnp.int32)
    win1 = m1 < acc
    acc = jnp.where(win1, _round_f32_to_bf16_rte(m1), acc)
    chsel = jnp.where(win1, 1, chsel)
    win2 = m2 < acc
    chsel = jnp.where(win2, 2, chsel)

    krf = krf_ref[...]
    fK = jnp.float32(K)

    def _first(seg, mc, kseg):
        return jnp.min(jnp.where(seg == mc[:, None], kseg, fK), axis=1)

    k_col0 = krf[:, _B0:_B0 + 128]
    k_col1 = krf[:, _B1:_B1 + 128]
    i0 = jnp.minimum(
        _first(s_ref[:, 0:_B0], m0, krf[:, 0:_B0]),
        jnp.min(jnp.where((li < 48) & (col0 == m0[:, None]), k_col0, fK),
                axis=1))
    i1 = jnp.minimum(
        _first(s_ref[:, _B0 + 128:_B1], m1, krf[:, _B0 + 128:_B1]),
        jnp.minimum(
            jnp.min(jnp.where((li >= 48) & (col0 == m1[:, None]), k_col0, fK),
                    axis=1),
            jnp.min(jnp.where((li < 96) & (col1 == m1[:, None]), k_col1, fK),
                    axis=1)))
    i2 = jnp.minimum(
        _first(s_ref[:, _B1 + 128:K], m2, krf[:, _B1 + 128:K]),
        jnp.min(jnp.where((li >= 96) & (col1 == m2[:, None]), k_col1, fK),
                axis=1))

    ind = jnp.where(chsel == 1, i1, i0)
    ind = jnp.where(chsel == 2, i2, ind)
    ind_ref[0, 0, :] = ind.astype(jnp.int32)


def _argmax_indices(x_flat, embed_t, a2, c2, krf):
    rows = x_flat.shape[0]
    grid = rows // TN
    return pl.pallas_call(
        _dist_argmax_kernel,
        grid=(grid,),
        in_specs=[
            pl.BlockSpec((TN, D), lambda i: (i, 0)),
            pl.BlockSpec((D, K), lambda i: (0, 0)),
            pl.BlockSpec((TN, 1), lambda i: (i, 0)),
            pl.BlockSpec((1, K), lambda i: (0, 0)),
            pl.BlockSpec((1, K), lambda i: (0, 0)),
        ],
        out_specs=pl.BlockSpec((1, 1, TN), lambda i: (i, 0, 0)),
        out_shape=jax.ShapeDtypeStruct((grid, 1, TN), jnp.int32),
        scratch_shapes=[pltpu.VMEM((TN, K), jnp.float32)],
        compiler_params=pltpu.CompilerParams(
            dimension_semantics=("arbitrary",)),
    )(x_flat, embed_t, a2, c2, krf)


_SC_INFO = plsc.get_sparse_core_info()
_NC = _SC_INFO.num_cores
_NS = _SC_INFO.num_subcores
_NW = _NC * _NS
_CS = 128

_gather_mesh = plsc.VectorSubcoreMesh(core_axis_name="c", subcore_axis_name="s")


def _make_gather_sc(rows):
    bpw = rows // _NW
    nch = bpw // _CS

    @functools.partial(
        pl.kernel,
        mesh=_gather_mesh,
        out_type=jax.ShapeDtypeStruct((rows, D), jnp.float32),
        scratch_types=[
            pltpu.VMEM((nch, _CS), jnp.int32),
            pltpu.VMEM((_CS, D), jnp.float32),
            pltpu.VMEM((_CS, D), jnp.float32),
            pltpu.SemaphoreType.DMA,
            pltpu.SemaphoreType.DMA,
        ],
    )
    def gather(table_hbm, idx_hbm, out_hbm, idx_v, rows0, rows1, sem0, sem1):
        wid = lax.axis_index("s") * _NC + lax.axis_index("c")
        base = wid * bpw
        pltpu.sync_copy(idx_hbm.at[wid], idx_v)
        bufs = (rows0, rows1)
        sems = (sem0, sem1)
        pltpu.async_copy(table_hbm.at[idx_v.at[0]], bufs[0], sems[0])
        for ch in range(nch):
            pltpu.make_async_copy(table_hbm.at[idx_v.at[ch]], bufs[ch % 2],
                                  sems[ch % 2]).wait()
            if ch + 1 < nch:
                pltpu.async_copy(table_hbm.at[idx_v.at[ch + 1]],
                                 bufs[(ch + 1) % 2], sems[(ch + 1) % 2])
            pltpu.sync_copy(bufs[ch % 2],
                            out_hbm.at[pl.ds(base + ch * _CS, _CS)])

    return gather, nch


_HALF = N // 2
_gather_half, _NCH_HALF = _make_gather_sc(_HALF)


def kernel(x, embed):
    shape = x.shape
    x_flat = x.reshape(-1, D)
    embed_t = embed.T
    a = jnp.sum(x_flat ** 2, axis=1, keepdims=True)
    c = jnp.sum(embed_t ** 2, axis=0, keepdims=True)
    kr = lax.broadcasted_iota(jnp.int32, (1, K), 1)
    krf = kr.astype(jnp.float32)
    xb = x_flat.astype(jnp.bfloat16)
    eb = embed_t.astype(jnp.bfloat16)
    a2 = a * 0.5
    c2 = c * 0.5
    ind0 = _argmax_indices(xb[:_HALF], eb, a2[:_HALF], c2, krf)
    ind1 = _argmax_indices(xb[_HALF:], eb, a2[_HALF:], c2, krf)
    i0 = ind0.reshape(_HALF)
    i1 = ind1.reshape(_HALF)
    q0 = _gather_half(embed, i0.reshape(_NW, _NCH_HALF, _CS))
    q1 = _gather_half(embed, i1.reshape(_NW, _NCH_HALF, _CS))
    quantized = jnp.concatenate([q0, q1], axis=0)
    ind_flat = jnp.concatenate([i0, i1], axis=0)
    return quantized.reshape(shape), ind_flat.reshape(shape[:-1])


# speedup vs baseline: 1.1486x; 1.1486x over previous
import functools

import jax
import jax.numpy as jnp
from jax import lax
from jax.experimental import pallas as pl
from jax.experimental.pallas import tpu as pltpu
from jax.experimental.pallas import tpu_sc as plsc

N = 16384
K = 8192
D = 256
TN = 512
GRID = N // TN

_CHUNK = 2736
_B0 = 21 * 128
_B1 = 42 * 128
_INF = float("inf")


def _round_f32_to_bf16_rte(v):
    u = lax.bitcast_convert_type(v, jnp.uint32)
    r = (u + jnp.uint32(0x7FFF) + ((u >> jnp.uint32(16)) & jnp.uint32(1))) \
        & jnp.uint32(0xFFFF0000)
    return lax.bitcast_convert_type(r, jnp.float32)


def _lane_iota(tn):
    return lax.broadcasted_iota(jnp.int32, (tn, 128), 1)


def _dist_argmax_kernel(x_ref, et_ref, a2_ref, c2_ref, krf_ref,
                        ind_ref, s_ref):
    x = x_ref[...]
    e = et_ref[...]
    b = lax.dot_general(x, e, (((1,), (1,)), ((), ())),
                        preferred_element_type=jnp.float32)
    s_ref[...] = (a2_ref[...] - b) + c2_ref[...]

    li = _lane_iota(TN)
    col0 = s_ref[:, _B0:_B0 + 128]
    col1 = s_ref[:, _B1:_B1 + 128]
    m0 = jnp.minimum(
        jnp.min(s_ref[:, 0:_B0], axis=1),
        jnp.min(jnp.where(li < 48, col0, _INF), axis=1))
    m1 = jnp.minimum(
        jnp.min(s_ref[:, _B0 + 128:_B1], axis=1),
        jnp.minimum(
            jnp.min(jnp.where(li >= 48, col0, _INF), axis=1),
            jnp.min(jnp.where(li < 96, col1, _INF), axis=1)))
    m2 = jnp.minimum(
        jnp.min(s_ref[:, _B1 + 128:K], axis=1),
        jnp.min(jnp.where(li >= 96, col1, _INF), axis=1))

    acc = _round_f32_to_bf16_rte(m0)
    chsel = jnp.zeros(m0.shape, jnp.int32)
    win1 = m1 < acc
    acc = jnp.where(win1, _round_f32_to_bf16_rte(m1), acc)
    chsel = jnp.where(win1, 1, chsel)
    win2 = m2 < acc
    chsel = jnp.where(win2, 2, chsel)

    krf = krf_ref[...]
    fK = jnp.float32(K)

    def _first(seg, mc, kseg):
        return jnp.min(jnp.where(seg == mc[:, None], kseg, fK), axis=1)

    k_col0 = krf[:, _B0:_B0 + 128]
    k_col1 = krf[:, _B1:_B1 + 128]
    i0 = jnp.minimum(
        _first(s_ref[:, 0:_B0], m0, krf[:, 0:_B0]),
        jnp.min(jnp.where((li < 48) & (col0 == m0[:, None]), k_col0, fK),
                axis=1))
    i1 = jnp.minimum(
        _first(s_ref[:, _B0 + 128:_B1], m1, krf[:, _B0 + 128:_B1]),
        jnp.minimum(
            jnp.min(jnp.where((li >= 48) & (col0 == m1[:, None]), k_col0, fK),
                    axis=1),
            jnp.min(jnp.where((li < 96) & (col1 == m1[:, None]), k_col1, fK),
                    axis=1)))
    i2 = jnp.minimum(
        _first(s_ref[:, _B1 + 128:K], m2, krf[:, _B1 + 128:K]),
        jnp.min(jnp.where((li >= 96) & (col1 == m2[:, None]), k_col1, fK),
                axis=1))

    ind = jnp.where(chsel == 1, i1, i0)
    ind = jnp.where(chsel == 2, i2, ind)
    ind_ref[0, 0, :] = ind.astype(jnp.int32)


def _argmax_indices(x_flat, embed_t, a2, c2, krf):
    rows = x_flat.shape[0]
    grid = rows // TN
    return pl.pallas_call(
        _dist_argmax_kernel,
        grid=(grid,),
        in_specs=[
            pl.BlockSpec((TN, D), lambda i: (i, 0)),
            pl.BlockSpec((K, D), lambda i: (0, 0)),
            pl.BlockSpec((TN, 1), lambda i: (i, 0)),
            pl.BlockSpec((1, K), lambda i: (0, 0)),
            pl.BlockSpec((1, K), lambda i: (0, 0)),
        ],
        out_specs=pl.BlockSpec((1, 1, TN), lambda i: (i, 0, 0)),
        out_shape=jax.ShapeDtypeStruct((grid, 1, TN), jnp.int32),
        scratch_shapes=[pltpu.VMEM((TN, K), jnp.float32)],
        compiler_params=pltpu.CompilerParams(
            dimension_semantics=("arbitrary",)),
    )(x_flat, embed_t, a2, c2, krf)


_SC_INFO = plsc.get_sparse_core_info()
_NC = _SC_INFO.num_cores
_NS = _SC_INFO.num_subcores
_NW = _NC * _NS
_CS = 128

_gather_mesh = plsc.VectorSubcoreMesh(core_axis_name="c", subcore_axis_name="s")


def _make_gather_sc(rows):
    bpw = rows // _NW
    nch = bpw // _CS

    @functools.partial(
        pl.kernel,
        mesh=_gather_mesh,
        out_type=jax.ShapeDtypeStruct((rows, D), jnp.float32),
        scratch_types=[
            pltpu.VMEM((nch, _CS), jnp.int32),
            pltpu.VMEM((_CS, D), jnp.float32),
            pltpu.VMEM((_CS, D), jnp.float32),
            pltpu.SemaphoreType.DMA,
            pltpu.SemaphoreType.DMA,
        ],
    )
    def gather(table_hbm, idx_hbm, out_hbm, idx_v, rows0, rows1, sem0, sem1):
        wid = lax.axis_index("s") * _NC + lax.axis_index("c")
        base = wid * bpw
        pltpu.sync_copy(idx_hbm.at[wid], idx_v)
        bufs = (rows0, rows1)
        sems = (sem0, sem1)
        pltpu.async_copy(table_hbm.at[idx_v.at[0]], bufs[0], sems[0])
        for ch in range(nch):
            pltpu.make_async_copy(table_hbm.at[idx_v.at[ch]], bufs[ch % 2],
                                  sems[ch % 2]).wait()
            if ch + 1 < nch:
                pltpu.async_copy(table_hbm.at[idx_v.at[ch + 1]],
                                 bufs[(ch + 1) % 2], sems[(ch + 1) % 2])
            pltpu.sync_copy(bufs[ch % 2],
                            out_hbm.at[pl.ds(base + ch * _CS, _CS)])

    return gather, nch


_gather_full, _NCH_FULL = _make_gather_sc(N)


def kernel(x, embed):
    shape = x.shape
    x_flat = x.reshape(-1, D)
    a = jnp.sum(x_flat ** 2, axis=1, keepdims=True)
    c = jnp.sum(embed ** 2, axis=1)[None, :]
    kr = lax.broadcasted_iota(jnp.int32, (1, K), 1)
    krf = kr.astype(jnp.float32)
    ind = _argmax_indices(x_flat.astype(jnp.bfloat16),
                          embed.astype(jnp.bfloat16),
                          a * 0.5, c * 0.5, krf)
    ind_flat = ind.reshape(N)
    quantized = _gather_full(embed, ind_flat.reshape(_NW, _NCH_FULL, _CS))
    return quantized.reshape(shape), ind_flat.reshape(shape[:-1])


# speedup vs baseline: 1.1900x; 1.0360x over previous
import functools

import jax
import jax.numpy as jnp
from jax import lax
from jax.experimental import pallas as pl
from jax.experimental.pallas import tpu as pltpu
from jax.experimental.pallas import tpu_sc as plsc

N = 16384
K = 8192
D = 256
TN = 1024
GRID = N // TN

_CHUNK = 2736
_B0 = 21 * 128
_B1 = 42 * 128
_INF = float("inf")


def _round_f32_to_bf16_rte(v):
    u = lax.bitcast_convert_type(v, jnp.uint32)
    r = (u + jnp.uint32(0x7FFF) + ((u >> jnp.uint32(16)) & jnp.uint32(1))) \
        & jnp.uint32(0xFFFF0000)
    return lax.bitcast_convert_type(r, jnp.float32)


def _lane_iota(tn):
    return lax.broadcasted_iota(jnp.int32, (tn, 128), 1)


def _dist_argmax_kernel(x_ref, et_ref, a2_ref, c2_ref, krf_ref,
                        ind_ref, s_ref):
    x = x_ref[...]
    e = et_ref[...]
    b = lax.dot_general(x, e, (((1,), (1,)), ((), ())),
                        preferred_element_type=jnp.float32)
    s_ref[...] = (a2_ref[...] - b) + c2_ref[...]

    li = _lane_iota(TN)
    col0 = s_ref[:, _B0:_B0 + 128]
    col1 = s_ref[:, _B1:_B1 + 128]
    m0 = jnp.minimum(
        jnp.min(s_ref[:, 0:_B0], axis=1),
        jnp.min(jnp.where(li < 48, col0, _INF), axis=1))
    m1 = jnp.minimum(
        jnp.min(s_ref[:, _B0 + 128:_B1], axis=1),
        jnp.minimum(
            jnp.min(jnp.where(li >= 48, col0, _INF), axis=1),
            jnp.min(jnp.where(li < 96, col1, _INF), axis=1)))
    m2 = jnp.minimum(
        jnp.min(s_ref[:, _B1 + 128:K], axis=1),
        jnp.min(jnp.where(li >= 96, col1, _INF), axis=1))

    acc = _round_f32_to_bf16_rte(m0)
    chsel = jnp.zeros(m0.shape, jnp.int32)
    win1 = m1 < acc
    acc = jnp.where(win1, _round_f32_to_bf16_rte(m1), acc)
    chsel = jnp.where(win1, 1, chsel)
    win2 = m2 < acc
    chsel = jnp.where(win2, 2, chsel)

    krf = krf_ref[...]
    fK = jnp.float32(K)

    def _first(seg, mc, kseg):
        return jnp.min(jnp.where(seg == mc[:, None], kseg, fK), axis=1)

    k_col0 = krf[:, _B0:_B0 + 128]
    k_col1 = krf[:, _B1:_B1 + 128]
    i0 = jnp.minimum(
        _first(s_ref[:, 0:_B0], m0, krf[:, 0:_B0]),
        jnp.min(jnp.where((li < 48) & (col0 == m0[:, None]), k_col0, fK),
                axis=1))
    i1 = jnp.minimum(
        _first(s_ref[:, _B0 + 128:_B1], m1, krf[:, _B0 + 128:_B1]),
        jnp.minimum(
            jnp.min(jnp.where((li >= 48) & (col0 == m1[:, None]), k_col0, fK),
                    axis=1),
            jnp.min(jnp.where((li < 96) & (col1 == m1[:, None]), k_col1, fK),
                    axis=1)))
    i2 = jnp.minimum(
        _first(s_ref[:, _B1 + 128:K], m2, krf[:, _B1 + 128:K]),
        jnp.min(jnp.where((li >= 96) & (col1 == m2[:, None]), k_col1, fK),
                axis=1))

    ind = jnp.where(chsel == 1, i1, i0)
    ind = jnp.where(chsel == 2, i2, ind)
    ind_ref[0, 0, :] = ind.astype(jnp.int32)


def _argmax_indices(x_flat, embed_t, a2, c2, krf):
    rows = x_flat.shape[0]
    grid = rows // TN
    return pl.pallas_call(
        _dist_argmax_kernel,
        grid=(grid,),
        in_specs=[
            pl.BlockSpec((TN, D), lambda i: (i, 0)),
            pl.BlockSpec((K, D), lambda i: (0, 0)),
            pl.BlockSpec((TN, 1), lambda i: (i, 0)),
            pl.BlockSpec((1, K), lambda i: (0, 0)),
            pl.BlockSpec((1, K), lambda i: (0, 0)),
        ],
        out_specs=pl.BlockSpec((1, 1, TN), lambda i: (i, 0, 0)),
        out_shape=jax.ShapeDtypeStruct((grid, 1, TN), jnp.int32),
        scratch_shapes=[pltpu.VMEM((TN, K), jnp.float32)],
        compiler_params=pltpu.CompilerParams(
            dimension_semantics=("arbitrary",)),
    )(x_flat, embed_t, a2, c2, krf)


_SC_INFO = plsc.get_sparse_core_info()
_NC = _SC_INFO.num_cores
_NS = _SC_INFO.num_subcores
_NW = _NC * _NS
_CS = 128

_gather_mesh = plsc.VectorSubcoreMesh(core_axis_name="c", subcore_axis_name="s")


def _make_gather_sc(rows):
    bpw = rows // _NW
    nch = bpw // _CS

    @functools.partial(
        pl.kernel,
        mesh=_gather_mesh,
        out_type=jax.ShapeDtypeStruct((rows, D), jnp.float32),
        scratch_types=[
            pltpu.VMEM((nch, _CS), jnp.int32),
            pltpu.VMEM((_CS, D), jnp.float32),
            pltpu.VMEM((_CS, D), jnp.float32),
            pltpu.SemaphoreType.DMA,
            pltpu.SemaphoreType.DMA,
        ],
    )
    def gather(table_hbm, idx_hbm, out_hbm, idx_v, rows0, rows1, sem0, sem1):
        wid = lax.axis_index("s") * _NC + lax.axis_index("c")
        base = wid * bpw
        pltpu.sync_copy(idx_hbm.at[wid], idx_v)
        bufs = (rows0, rows1)
        sems = (sem0, sem1)
        pltpu.async_copy(table_hbm.at[idx_v.at[0]], bufs[0], sems[0])
        for ch in range(nch):
            pltpu.make_async_copy(table_hbm.at[idx_v.at[ch]], bufs[ch % 2],
                                  sems[ch % 2]).wait()
            if ch + 1 < nch:
                pltpu.async_copy(table_hbm.at[idx_v.at[ch + 1]],
                                 bufs[(ch + 1) % 2], sems[(ch + 1) % 2])
            pltpu.sync_copy(bufs[ch % 2],
                            out_hbm.at[pl.ds(base + ch * _CS, _CS)])

    return gather, nch


_gather_full, _NCH_FULL = _make_gather_sc(N)


def kernel(x, embed):
    shape = x.shape
    x_flat = x.reshape(-1, D)
    a = jnp.sum(x_flat ** 2, axis=1, keepdims=True)
    c = jnp.sum(embed ** 2, axis=1)[None, :]
    kr = lax.broadcasted_iota(jnp.int32, (1, K), 1)
    krf = kr.astype(jnp.float32)
    ind = _argmax_indices(x_flat.astype(jnp.bfloat16),
                          embed.astype(jnp.bfloat16),
                          a * 0.5, c * 0.5, krf)
    ind_flat = ind.reshape(N)
    quantized = _gather_full(embed, ind_flat.reshape(_NW, _NCH_FULL, _CS))
    return quantized.reshape(shape), ind_flat.reshape(shape[:-1])
